# baseline (speedup 1.0000x reference)
import functools

import jax
import jax.numpy as jnp
from jax import lax
from jax.experimental import pallas as pl
from jax.experimental.pallas import tpu as pltpu
from jax.experimental.pallas import tpu_sc as plsc

_NUM_CORES = 2
_NUM_SUBCORES = 16
_NUM_WORKERS = _NUM_CORES * _NUM_SUBCORES
_LANES = 16


@functools.partial(jax.jit, static_argnames=("batch", "seq_len"))
def _embed_sc(ids_flat, wte, wpe, *, batch, seq_len):
    n_embd = wte.shape[1]
    k = seq_len // _NUM_WORKERS
    lanes_per_row = n_embd // _LANES

    mesh = plsc.VectorSubcoreMesh(
        core_axis_name="c",
        subcore_axis_name="s",
        num_cores=_NUM_CORES,
        num_subcores=_NUM_SUBCORES,
    )

    @functools.partial(
        pl.kernel,
        out_type=jax.ShapeDtypeStruct((batch * seq_len, n_embd), jnp.float32),
        mesh=mesh,
        scratch_types=[
            pltpu.VMEM((batch * k,), jnp.int32),
            pltpu.VMEM((k, n_embd), jnp.float32),
            pltpu.VMEM((k, n_embd), jnp.float32),
            pltpu.SemaphoreType.DMA,
            pltpu.SemaphoreType.DMA,
        ],
    )
    def body(ids_hbm, wte_hbm, wpe_hbm, out_hbm, idx_v, rows_v, wpe_v,
             sem_g, sem_p):
        wid = lax.axis_index("s") * _NUM_CORES + lax.axis_index("c")
        s0 = wid * k

        wpe_load = pltpu.async_copy(wpe_hbm.at[pl.ds(s0, k), :], wpe_v, sem_p)
        id_loads = [
            pltpu.async_copy(
                ids_hbm.at[pl.ds(b * seq_len + s0, k)],
                idx_v.at[pl.ds(b * k, k)],
                sem_g,
            )
            for b in range(batch)
        ]
        id_loads[0].wait()
        gather = pltpu.async_copy(
            wte_hbm.at[idx_v.at[pl.ds(0, k)]], rows_v, sem_g
        )
        for ld in id_loads[1:]:
            ld.wait()
        wpe_load.wait()

        h = k // 2
        for b in range(batch):
            base = b * seq_len + s0
            gather.wait()

            def add_row(r):
                for j in range(lanes_per_row):
                    sl = pl.ds(j * _LANES, _LANES)
                    rows_v[r, sl] += wpe_v[r, sl]

            pl.loop(0, h)(add_row)
            st_a = pltpu.async_copy(
                rows_v.at[pl.ds(0, h)], out_hbm.at[pl.ds(base, h), :], sem_p
            )
            pl.loop(h, k)(add_row)
            pltpu.sync_copy(
                rows_v.at[pl.ds(h, h)], out_hbm.at[pl.ds(base + h, h), :]
            )
            st_a.wait()
            if b + 1 < batch:
                gather = pltpu.async_copy(
                    wte_hbm.at[idx_v.at[pl.ds((b + 1) * k, k)]], rows_v, sem_g
                )

    return body(ids_flat, wte, wpe)


def kernel(input_ids, wte, wpe):
    batch, seq_len = input_ids.shape
    out = _embed_sc(input_ids.reshape(-1), wte, wpe, batch=batch, seq_len=seq_len)
    return out.reshape(batch, seq_len, wte.shape[1])


# speedup vs baseline: 1.3693x; 1.3693x over previous
import functools

import jax
import jax.numpy as jnp
from jax import lax
from jax.experimental import pallas as pl
from jax.experimental.pallas import tpu as pltpu
from jax.experimental.pallas import tpu_sc as plsc

_NUM_CORES = 2
_NUM_SUBCORES = 16
_NUM_WORKERS = _NUM_CORES * _NUM_SUBCORES
_LANES = 16


@functools.partial(jax.jit, static_argnames=("batch", "seq_len"))
def _embed_sc(ids_flat, wte, wpe, *, batch, seq_len):
    n_embd = wte.shape[1]
    k = seq_len // _NUM_WORKERS
    lanes_per_row = n_embd // _LANES

    mesh = plsc.VectorSubcoreMesh(
        core_axis_name="c",
        subcore_axis_name="s",
        num_cores=_NUM_CORES,
        num_subcores=_NUM_SUBCORES,
    )

    @functools.partial(
        pl.kernel,
        out_type=jax.ShapeDtypeStruct((batch * seq_len, n_embd), jnp.float32),
        mesh=mesh,
        scratch_types=[
            pltpu.VMEM((batch * k,), jnp.int32),
            pltpu.VMEM((k, n_embd), jnp.float32),
            pltpu.VMEM((k, n_embd), jnp.float32),
            pltpu.SemaphoreType.DMA,
            pltpu.SemaphoreType.DMA,
        ],
    )
    def body(ids_hbm, wte_hbm, wpe_hbm, out_hbm, idx_v, rows_v, wpe_v,
             sem_g, sem_p):
        wid = lax.axis_index("s") * _NUM_CORES + lax.axis_index("c")
        s0 = wid * k

        wpe_load = pltpu.async_copy(wpe_hbm.at[pl.ds(s0, k), :], wpe_v, sem_p)
        id_loads = [
            pltpu.async_copy(
                ids_hbm.at[pl.ds(b * seq_len + s0, k)],
                idx_v.at[pl.ds(b * k, k)],
                sem_g,
            )
            for b in range(batch)
        ]
        id_loads[0].wait()
        gather = pltpu.async_copy(
            wte_hbm.at[idx_v.at[pl.ds(0, k)]], rows_v, sem_g
        )
        for ld in id_loads[1:]:
            ld.wait()
        wpe_load.wait()

        for b in range(batch):
            base = b * seq_len + s0
            gather.wait()

            def add_row(r):
                for j in range(lanes_per_row):
                    sl = pl.ds(j * _LANES, _LANES)
                    rows_v[r, sl] += wpe_v[r, sl]

            pl.loop(0, k)(add_row)
            pltpu.sync_copy(rows_v, out_hbm.at[pl.ds(base, k), :])
            if b + 1 < batch:
                gather = pltpu.async_copy(
                    wte_hbm.at[idx_v.at[pl.ds((b + 1) * k, k)]], rows_v, sem_g
                )

    return body(ids_flat, wte, wpe)


def kernel(input_ids, wte, wpe):
    batch, seq_len = input_ids.shape
    out = _embed_sc(input_ids.reshape(-1), wte, wpe, batch=batch, seq_len=seq_len)
    return out.reshape(batch, seq_len, wte.shape[1])
